# baseline (speedup 1.0000x reference)
import functools

import jax
import jax.numpy as jnp
from jax import lax
from jax.experimental import pallas as pl
from jax.experimental.pallas import tpu as pltpu
from jax.experimental.pallas import tpu_sc as plsc

_H = 224
_W = 224
_F = 4
_DSUB = 128
_D = 3 * _DSUB
_LANES = 16
_VJ = _DSUB // _LANES
_NWORKERS = 32
_SLABS = _F * _H
_PER_W = _SLABS // _NWORKERS
_QROWS = 56
_NQ = _W // _QROWS
_NBUF = 3


def _materialize(cw, rw, fw):
    mesh = plsc.VectorSubcoreMesh(core_axis_name="c", subcore_axis_name="s")

    @functools.partial(
        pl.kernel,
        mesh=mesh,
        out_type=jax.ShapeDtypeStruct((_F, _H, _W, _D), jnp.float32),
        scratch_types=[
            pltpu.VMEM((_QROWS, _D), jnp.float32),
            pltpu.VMEM((_QROWS, _D), jnp.float32),
            pltpu.VMEM((_QROWS, _D), jnp.float32),
            pltpu.VMEM((_W, _DSUB), jnp.float32),
            pltpu.VMEM((32, _DSUB), jnp.float32),
            pltpu.VMEM((_F, _DSUB), jnp.float32),
            pltpu.SemaphoreType.DMA,
            pltpu.SemaphoreType.DMA,
            pltpu.SemaphoreType.DMA,
        ],
    )
    def kern(cw_hbm, rw_hbm, fw_hbm, x_hbm, buf0, buf1, buf2, rbuf, crows,
             fstage, sem0, sem1, sem2):
        wid = lax.axis_index("s") * 2 + lax.axis_index("c")
        per_k = _H // _PER_W
        kk = wid // per_k
        ibase = (wid % per_k) * _PER_W
        astart = pl.multiple_of((ibase // 8) * 8, 8)
        aoff = ibase - astart

        pltpu.sync_copy(rw_hbm.at[pl.ds(0, _W)], rbuf)
        pltpu.sync_copy(cw_hbm.at[pl.ds(astart, 32)], crows)
        pltpu.sync_copy(fw_hbm.at[pl.ds(0, _F)], fstage)

        bufs = (buf0, buf1, buf2)
        sems = (sem0, sem1, sem2)

        def quarter(q, buf, sem):
            p = q // _NQ
            r = q % _NQ
            ii = ibase + p
            jlo = r * _QROWS
            dst = x_hbm.at[kk, ii, pl.ds(jlo, _QROWS), :]

            @pl.when(q >= _NBUF)
            def _():
                pltpu.make_async_copy(buf, dst, sem).wait()

            cvecs = [crows[aoff + p, pl.ds(m * _LANES, _LANES)]
                     for m in range(_VJ)]
            fvecs = [fstage[kk, pl.ds(m * _LANES, _LANES)]
                     for m in range(_VJ)]

            def row(j, carry):
                for m in range(_VJ):
                    buf[j, pl.ds(m * _LANES, _LANES)] = cvecs[m]
                for m in range(_VJ):
                    buf[j, pl.ds(_DSUB + m * _LANES, _LANES)] = (
                        rbuf[jlo + j, pl.ds(m * _LANES, _LANES)])
                for m in range(_VJ):
                    buf[j, pl.ds(2 * _DSUB + m * _LANES, _LANES)] = fvecs[m]
                return carry

            lax.fori_loop(0, _QROWS, row, 0)
            pltpu.make_async_copy(buf, dst, sem).start()

        def step(q, carry):
            for b in range(_NBUF):
                @pl.when(q % _NBUF == b)
                def _():
                    quarter(q, bufs[b], sems[b])
            return carry

        lax.fori_loop(0, _PER_W * _NQ, step, 0)

        for b in range(_NBUF):
            pltpu.make_async_copy(
                bufs[b], x_hbm.at[kk, ibase, pl.ds(0, _QROWS), :],
                sems[b]).wait()

    return kern(cw, rw, fw)


def kernel(patch, num_views, row_weight, col_weight, frame_weight):
    x = _materialize(col_weight, row_weight, frame_weight)
    return jnp.transpose(x, (0, 3, 1, 2))


# speedup vs baseline: 2.0081x; 2.0081x over previous
import functools

import jax
import jax.numpy as jnp
from jax import lax
from jax.experimental import pallas as pl
from jax.experimental.pallas import tpu as pltpu
from jax.experimental.pallas import tpu_sc as plsc

_H = 224
_W = 224
_HW = _W // 2
_F = 4
_DSUB = 128
_LANES = 16
_VJ = _DSUB // _LANES
_NWORKERS = 32
_SLABS = _F * _H
_PER_W = _SLABS // _NWORKERS


def _materialize(cw, rw, fw):
    mesh = plsc.VectorSubcoreMesh(core_axis_name="c", subcore_axis_name="s")

    @functools.partial(
        pl.kernel,
        mesh=mesh,
        out_type=jax.ShapeDtypeStruct((_F, _H, _W, 3 * _DSUB), jnp.float32),
        scratch_types=[
            pltpu.VMEM((_HW, _DSUB), jnp.float32),
            pltpu.VMEM((_HW, _DSUB), jnp.float32),
            pltpu.VMEM((_W, _DSUB), jnp.float32),
            pltpu.VMEM((_HW, _DSUB), jnp.float32),
            pltpu.VMEM((_H, _DSUB), jnp.float32),
            pltpu.VMEM((_F, _DSUB), jnp.float32),
            pltpu.SemaphoreType.DMA,
            pltpu.SemaphoreType.DMA,
            pltpu.SemaphoreType.DMA,
        ],
    )
    def kern(cw_hbm, rw_hbm, fw_hbm, x_hbm, cbuf_a, cbuf_b, rbuf, fbuf,
             cstage, fstage, sem_a, sem_b, sem_rf):
        wid = lax.axis_index("s") * 2 + lax.axis_index("c")
        per_k = _H // _PER_W
        kk = wid // per_k
        ibase = (wid % per_k) * _PER_W

        stage = [
            pltpu.make_async_copy(rw_hbm.at[pl.ds(0, _W)], rbuf, sem_a),
            pltpu.make_async_copy(cw_hbm.at[pl.ds(0, _H)], cstage, sem_a),
            pltpu.make_async_copy(fw_hbm.at[pl.ds(0, _F)], fstage, sem_a),
        ]
        for c in stage:
            c.start()
        for c in stage:
            c.wait()

        def fill(buf, vecs):
            def row(j, carry):
                for m in range(_VJ):
                    buf[j, pl.ds(m * _LANES, _LANES)] = vecs[m]
                return carry

            lax.fori_loop(0, _HW, row, 0)

        def start2(buf, ii, dlo, sem):
            for jlo in (0, _HW):
                pltpu.make_async_copy(
                    buf, x_hbm.at[kk, ii, pl.ds(jlo, _HW), pl.ds(dlo, _DSUB)],
                    sem).start()

        def wait2(buf, dlo, sem):
            for jlo in (0, _HW):
                pltpu.make_async_copy(
                    buf, x_hbm.at[kk, ibase, pl.ds(jlo, _HW),
                                  pl.ds(dlo, _DSUB)], sem).wait()

        fill(fbuf, [fstage[kk, pl.ds(m * _LANES, _LANES)] for m in range(_VJ)])

        def step(p, carry):
            ii = ibase + p

            pltpu.make_async_copy(
                rbuf, x_hbm.at[kk, ii, :, pl.ds(_DSUB, _DSUB)], sem_rf).start()
            start2(fbuf, ii, 2 * _DSUB, sem_rf)

            @pl.when(p % 2 == 0)
            def _():
                @pl.when(p >= 2)
                def _():
                    wait2(cbuf_a, 0, sem_a)

                fill(cbuf_a, [cstage[ii, pl.ds(m * _LANES, _LANES)]
                              for m in range(_VJ)])
                start2(cbuf_a, ii, 0, sem_a)

            @pl.when(p % 2 == 1)
            def _():
                @pl.when(p >= 3)
                def _():
                    wait2(cbuf_b, 0, sem_b)

                fill(cbuf_b, [cstage[ii, pl.ds(m * _LANES, _LANES)]
                              for m in range(_VJ)])
                start2(cbuf_b, ii, 0, sem_b)

            return carry

        lax.fori_loop(0, _PER_W, step, 0)

        wait2(cbuf_a, 0, sem_a)
        wait2(cbuf_b, 0, sem_b)

        def drain(p, carry):
            pltpu.make_async_copy(
                rbuf, x_hbm.at[kk, ibase, :, pl.ds(_DSUB, _DSUB)],
                sem_rf).wait()
            wait2(fbuf, 2 * _DSUB, sem_rf)
            return carry

        lax.fori_loop(0, _PER_W, drain, 0)

    return kern(cw, rw, fw)


def kernel(patch, num_views, row_weight, col_weight, frame_weight):
    x = _materialize(col_weight, row_weight, frame_weight)
    return jnp.transpose(x, (0, 3, 1, 2))
